# Initial kernel scaffold; baseline (speedup 1.0000x reference)
import functools

import jax
import jax.numpy as jnp
from jax import lax
from jax.experimental import pallas as pl
from jax.experimental.pallas import tpu as pltpu
from jax.experimental.pallas import tpu_sc as plsc

_N = 10000
_E = 320000
_NC = 2
_NS = 16
_B = 80
_RPS = _N // _NS
_EBLK = _E // _B


def _make_deg_kernel():
    mesh = plsc.VectorSubcoreMesh(core_axis_name="c", subcore_axis_name="s")
    nblk = _EBLK // (_NC * _NS)

    @functools.partial(
        pl.kernel,
        out_type=jax.ShapeDtypeStruct((_NC * _N, 16), jnp.float32),
        mesh=mesh,
        scratch_types=[
            pltpu.VMEM((nblk, _B), jnp.int32),
            pltpu.VMEM((_B, 16), jnp.float32),
            pltpu.VMEM_SHARED((_N, 16), jnp.float32),
        ],
    )
    def deg_kernel(dst_hbm, ones_hbm, zeros_hbm, out_hbm, dstv, onesv, acc):
        c = lax.axis_index("c")
        s = lax.axis_index("s")
        w = c * _NS + s
        pltpu.sync_copy(dst_hbm.at[pl.ds(w * nblk, nblk)], dstv)
        pltpu.sync_copy(ones_hbm, onesv)
        pltpu.sync_copy(zeros_hbm.at[pl.ds(s * _RPS, _RPS)],
                        acc.at[pl.ds(s * _RPS, _RPS)])
        plsc.subcore_barrier()

        @pl.loop(0, nblk)
        def _(j):
            pltpu.sync_copy(onesv, acc.at[dstv.at[j]], add=True)

        plsc.subcore_barrier()
        pltpu.sync_copy(acc.at[pl.ds(s * _RPS, _RPS)],
                        out_hbm.at[pl.ds(c * _N + s * _RPS, _RPS)])

    return deg_kernel


def _make_agg_kernel(dh):
    mesh = plsc.VectorSubcoreMesh(core_axis_name="c", subcore_axis_name="s")
    nblk = _EBLK // _NS

    @functools.partial(
        pl.kernel,
        out_type=jax.ShapeDtypeStruct((_NC * _N, dh), jnp.float32),
        mesh=mesh,
        scratch_types=[
            pltpu.VMEM((nblk, _B), jnp.int32),
            pltpu.VMEM((nblk, _B), jnp.int32),
            pltpu.VMEM((_B, dh), jnp.float32),
            pltpu.VMEM_SHARED((_N, dh), jnp.float32),
        ],
    )
    def agg_kernel(h_hbm, src_hbm, dst_hbm, out_hbm, srcv, dstv, rows, acc):
        c = lax.axis_index("c")
        s = lax.axis_index("s")
        pltpu.sync_copy(src_hbm.at[pl.ds(c * _EBLK + s * nblk, nblk)], srcv)
        pltpu.sync_copy(dst_hbm.at[pl.ds(s * nblk, nblk)], dstv)
        pltpu.sync_copy(h_hbm.at[pl.ds(c * _N + s * _RPS, _RPS)],
                        acc.at[pl.ds(s * _RPS, _RPS)])
        plsc.subcore_barrier()

        @pl.loop(0, nblk)
        def _(j):
            pltpu.sync_copy(h_hbm.at[srcv.at[j]], rows)
            pltpu.sync_copy(rows, acc.at[dstv.at[j]], add=True)

        plsc.subcore_barrier()
        pltpu.sync_copy(acc.at[pl.ds(s * _RPS, _RPS)],
                        out_hbm.at[pl.ds(c * _N + s * _RPS, _RPS)])

    return agg_kernel


_deg = _make_deg_kernel()
_agg128 = _make_agg_kernel(128)
_agg64 = _make_agg_kernel(64)


def _mm_body(x_ref, w_ref, o_ref):
    o_ref[...] = jnp.dot(x_ref[...], w_ref[...],
                         preferred_element_type=jnp.float32)


def _mm(x, w):
    return pl.pallas_call(
        _mm_body,
        out_shape=jax.ShapeDtypeStruct((x.shape[0], w.shape[1]), jnp.float32),
    )(x, w)


def _scale_body(h_ref, cnt_ref, hcat_ref, d_ref):
    c0 = cnt_ref[0:_N, 0:1]
    c1 = cnt_ref[_N:, 0:1]
    d = lax.rsqrt(1.0 + c0 + c1)
    d_ref[...] = d
    hs = h_ref[...] * d
    hcat_ref[0:_N, :] = hs[:, 0:128]
    hcat_ref[_N:, :] = hs[:, 128:]


def _scale(h, cnt):
    return pl.pallas_call(
        _scale_body,
        out_shape=(jax.ShapeDtypeStruct((2 * _N, 128), jnp.float32),
                   jax.ShapeDtypeStruct((_N, 1), jnp.float32)),
    )(h, cnt)


def _mm2_body(a_ref, d_ref, b1_ref, w2_ref, o_ref):
    d = d_ref[...]
    h1lo = jnp.maximum(a_ref[0:_N, :] * d + b1_ref[0, 0:128], 0.0)
    h1hi = jnp.maximum(a_ref[_N:, :] * d + b1_ref[0, 128:], 0.0)
    h2 = (jnp.dot(h1lo, w2_ref[0:128, :], preferred_element_type=jnp.float32)
          + jnp.dot(h1hi, w2_ref[128:, :], preferred_element_type=jnp.float32))
    h2 = h2 * d
    o_ref[0:_N, :] = h2[:, 0:64]
    o_ref[_N:, :] = h2[:, 64:]


def _mm2(acc1, d, b1, w2):
    return pl.pallas_call(
        _mm2_body,
        out_shape=jax.ShapeDtypeStruct((2 * _N, 64), jnp.float32),
    )(acc1, d, b1, w2)


def _fin_body(a_ref, d_ref, b2_ref, o_ref):
    d = d_ref[...]
    o_ref[...] = (jnp.concatenate([a_ref[0:_N, :], a_ref[_N:, :]], axis=1) * d
                  + b2_ref[...])


def _fin(acc2, d, b2):
    return pl.pallas_call(
        _fin_body,
        out_shape=jax.ShapeDtypeStruct((_N, 128), jnp.float32),
    )(acc2, d, b2)


def kernel(x, edge_index, W1, b1, W2, b2):
    src = edge_index[0]
    dst = edge_index[1]
    dst2 = dst.reshape(_EBLK, _B)
    src2 = jnp.concatenate([src, src + _N]).reshape(2 * _EBLK, _B)
    ones16 = jnp.ones((_B, 16), jnp.float32)
    zeros16 = jnp.zeros((_N, 16), jnp.float32)

    cnt = _deg(dst2, ones16, zeros16)
    h = _mm(x, W1)
    hcat, d = _scale(h, cnt)
    acc1 = _agg128(hcat, src2, dst2)
    h2cat = _mm2(acc1, d, b1.reshape(1, -1), W2)
    acc2 = _agg64(h2cat, src2, dst2)
    return _fin(acc2, d, b2.reshape(1, -1))


# speedup vs baseline: 6.8967x; 6.8967x over previous
import functools

import jax
import jax.numpy as jnp
from jax import lax
from jax.experimental import pallas as pl
from jax.experimental.pallas import tpu as pltpu
from jax.experimental.pallas import tpu_sc as plsc

_N = 10000
_E = 320000
_NC = 2
_NS = 16
_B = 80
_NP = 10112
_RPS = _NP // _NS
_EP = 327680
_EBLK = _EP // _B
_DH = 64


def _make_deg_kernel():
    mesh = plsc.VectorSubcoreMesh(core_axis_name="c", subcore_axis_name="s")
    nblk = _EBLK // (_NC * _NS)

    @functools.partial(
        pl.kernel,
        out_type=jax.ShapeDtypeStruct((_NC * _NP, 16), jnp.float32),
        mesh=mesh,
        scratch_types=[
            pltpu.VMEM((nblk, _B), jnp.int32),
            pltpu.VMEM((_B, 16), jnp.float32),
            pltpu.VMEM_SHARED((_NP, 16), jnp.float32),
        ],
        compiler_params=pltpu.CompilerParams(use_tc_tiling_on_sc=False),
    )
    def deg_kernel(dst_hbm, ones_hbm, zeros_hbm, out_hbm, dstv, onesv, acc):
        c = lax.axis_index("c")
        s = lax.axis_index("s")
        w = c * _NS + s
        pltpu.sync_copy(dst_hbm.at[pl.ds(w * nblk, nblk)], dstv)
        pltpu.sync_copy(ones_hbm, onesv)
        pltpu.sync_copy(zeros_hbm.at[pl.ds(s * _RPS, _RPS)],
                        acc.at[pl.ds(s * _RPS, _RPS)])
        plsc.subcore_barrier()

        @pl.loop(0, nblk)
        def _(j):
            pltpu.sync_copy(onesv, acc.at[dstv.at[j]], add=True)

        plsc.subcore_barrier()
        pltpu.sync_copy(acc.at[pl.ds(s * _RPS, _RPS)],
                        out_hbm.at[pl.ds(c * _NP + s * _RPS, _RPS)])

    return deg_kernel


def _make_agg_kernel():
    mesh = plsc.VectorSubcoreMesh(core_axis_name="c", subcore_axis_name="s")
    nblk = _EBLK // _NS

    @functools.partial(
        pl.kernel,
        out_type=jax.ShapeDtypeStruct((_NC * _NP, _DH), jnp.float32),
        mesh=mesh,
        scratch_types=[
            pltpu.VMEM((nblk, _B), jnp.int32),
            pltpu.VMEM((nblk, _B), jnp.int32),
            pltpu.VMEM((_B, _DH), jnp.float32),
            pltpu.VMEM_SHARED((_NP, _DH), jnp.float32),
        ],
        compiler_params=pltpu.CompilerParams(use_tc_tiling_on_sc=False),
    )
    def agg_kernel(h_hbm, src_hbm, dst_hbm, out_hbm, srcv, dstv, rows, acc):
        c = lax.axis_index("c")
        s = lax.axis_index("s")
        pltpu.sync_copy(src_hbm.at[pl.ds(c * _EBLK + s * nblk, nblk)], srcv)
        pltpu.sync_copy(dst_hbm.at[pl.ds(s * nblk, nblk)], dstv)
        pltpu.sync_copy(h_hbm.at[pl.ds(c * _NP + s * _RPS, _RPS)],
                        acc.at[pl.ds(s * _RPS, _RPS)])
        plsc.subcore_barrier()

        @pl.loop(0, nblk)
        def _(j):
            pltpu.sync_copy(h_hbm.at[srcv.at[j]], rows)
            pltpu.sync_copy(rows, acc.at[dstv.at[j]], add=True)

        plsc.subcore_barrier()
        pltpu.sync_copy(acc.at[pl.ds(s * _RPS, _RPS)],
                        out_hbm.at[pl.ds(c * _NP + s * _RPS, _RPS)])

    return agg_kernel


_deg = _make_deg_kernel()
_agg = _make_agg_kernel()


def _mm_body(x_ref, w_ref, o_ref):
    o_ref[...] = jnp.dot(x_ref[...], w_ref[...],
                         preferred_element_type=jnp.float32)


def _mm(x, w):
    return pl.pallas_call(
        _mm_body,
        out_shape=jax.ShapeDtypeStruct((x.shape[0], w.shape[1]), jnp.float32),
    )(x, w)


def _scale_body(h_ref, cnt_ref, ha_ref, hb_ref, d_ref):
    c0 = cnt_ref[0:_N, 0:1]
    c1 = cnt_ref[_NP:_NP + _N, 0:1]
    d = lax.rsqrt(1.0 + c0 + c1)
    d_ref[...] = d
    hs = h_ref[...] * d
    ha_ref[0:_N, :] = hs[:, 0:64]
    ha_ref[_NP:_NP + _N, :] = hs[:, 64:128]
    hb_ref[0:_N, :] = hs[:, 128:192]
    hb_ref[_NP:_NP + _N, :] = hs[:, 192:256]


def _scale(h, cnt):
    return pl.pallas_call(
        _scale_body,
        out_shape=(jax.ShapeDtypeStruct((2 * _NP, _DH), jnp.float32),
                   jax.ShapeDtypeStruct((2 * _NP, _DH), jnp.float32),
                   jax.ShapeDtypeStruct((_N, 1), jnp.float32)),
    )(h, cnt)


def _mm2_body(aa_ref, ab_ref, d_ref, b1_ref, w2_ref, o_ref):
    d = d_ref[...]
    h1q0 = jnp.maximum(aa_ref[0:_N, :] * d + b1_ref[0, 0:64], 0.0)
    h1q1 = jnp.maximum(aa_ref[_NP:_NP + _N, :] * d + b1_ref[0, 64:128], 0.0)
    h1q2 = jnp.maximum(ab_ref[0:_N, :] * d + b1_ref[0, 128:192], 0.0)
    h1q3 = jnp.maximum(ab_ref[_NP:_NP + _N, :] * d + b1_ref[0, 192:256], 0.0)
    h2 = (jnp.dot(h1q0, w2_ref[0:64, :], preferred_element_type=jnp.float32)
          + jnp.dot(h1q1, w2_ref[64:128, :], preferred_element_type=jnp.float32)
          + jnp.dot(h1q2, w2_ref[128:192, :], preferred_element_type=jnp.float32)
          + jnp.dot(h1q3, w2_ref[192:256, :], preferred_element_type=jnp.float32))
    h2 = h2 * d
    o_ref[0:_N, :] = h2[:, 0:64]
    o_ref[_NP:_NP + _N, :] = h2[:, 64:]


def _mm2(a1a, a1b, d, b1, w2):
    return pl.pallas_call(
        _mm2_body,
        out_shape=jax.ShapeDtypeStruct((2 * _NP, _DH), jnp.float32),
    )(a1a, a1b, d, b1, w2)


def _fin_body(a_ref, d_ref, b2_ref, o_ref):
    d = d_ref[...]
    lo = a_ref[0:_N, :]
    hi = a_ref[_NP:_NP + _N, :]
    o_ref[...] = jnp.concatenate([lo, hi], axis=1) * d + b2_ref[...]


def _fin(acc2, d, b2):
    return pl.pallas_call(
        _fin_body,
        out_shape=jax.ShapeDtypeStruct((_N, 128), jnp.float32),
    )(acc2, d, b2)


def kernel(x, edge_index, W1, b1, W2, b2):
    src = edge_index[0]
    dst = edge_index[1]
    npad = _EP - _E
    srcp = jnp.concatenate([src, jnp.zeros((npad,), jnp.int32)])
    dstp = jnp.concatenate([dst, jnp.full((npad,), _N, jnp.int32)])
    dst2 = dstp.reshape(_EBLK, _B)
    src2 = jnp.concatenate([srcp, srcp + _NP]).reshape(2 * _EBLK, _B)
    ones16 = jnp.ones((_B, 16), jnp.float32)
    zeros16 = jnp.zeros((_NP, 16), jnp.float32)

    cnt = _deg(dst2, ones16, zeros16)
    h = _mm(x, W1)
    h1a, h1b, d = _scale(h, cnt)
    a1a = _agg(h1a, src2, dst2)
    a1b = _agg(h1b, src2, dst2)
    h2cat = _mm2(a1a, a1b, d, b1.reshape(1, -1), W2)
    acc2 = _agg(h2cat, src2, dst2)
    return _fin(acc2, d, b2.reshape(1, -1))


# speedup vs baseline: 9.7148x; 1.4086x over previous
import functools

import jax
import jax.numpy as jnp
from jax import lax
from jax.experimental import pallas as pl
from jax.experimental.pallas import tpu as pltpu
from jax.experimental.pallas import tpu_sc as plsc

_N = 10000
_E = 320000
_NC = 2
_NS = 16
_B = 128
_NP = 10112
_RPS = _NP // _NS
_EP = 327680
_EBLK = _EP // _B
_DH = 64
_K = 4


def _make_deg_kernel():
    mesh = plsc.VectorSubcoreMesh(core_axis_name="c", subcore_axis_name="s")
    nblk = _EBLK // (_NC * _NS)

    @functools.partial(
        pl.kernel,
        out_type=jax.ShapeDtypeStruct((_NC * _NP, 16), jnp.float32),
        mesh=mesh,
        scratch_types=[
            pltpu.VMEM((nblk, _B), jnp.int32),
            pltpu.VMEM((_B, 16), jnp.float32),
            pltpu.VMEM_SHARED((_NP, 16), jnp.float32),
        ],
        compiler_params=pltpu.CompilerParams(use_tc_tiling_on_sc=False),
    )
    def deg_kernel(dst_hbm, ones_hbm, zeros_hbm, out_hbm, dstv, onesv, acc):
        c = lax.axis_index("c")
        s = lax.axis_index("s")
        w = c * _NS + s
        pltpu.sync_copy(dst_hbm.at[pl.ds(w * nblk, nblk)], dstv)
        pltpu.sync_copy(ones_hbm, onesv)
        pltpu.sync_copy(zeros_hbm.at[pl.ds(s * _RPS, _RPS)],
                        acc.at[pl.ds(s * _RPS, _RPS)])
        plsc.subcore_barrier()

        @pl.loop(0, nblk)
        def _(j):
            pltpu.sync_copy(onesv, acc.at[dstv.at[j]], add=True)

        plsc.subcore_barrier()
        pltpu.sync_copy(acc.at[pl.ds(s * _RPS, _RPS)],
                        out_hbm.at[pl.ds(c * _NP + s * _RPS, _RPS)])

    return deg_kernel


def _make_agg_kernel():
    mesh = plsc.VectorSubcoreMesh(core_axis_name="c", subcore_axis_name="s")
    nblk = _EBLK // _NS

    @functools.partial(
        pl.kernel,
        out_type=jax.ShapeDtypeStruct((_NC * _NP, _DH), jnp.float32),
        mesh=mesh,
        scratch_types=[
            pltpu.VMEM((nblk, _B), jnp.int32),
            pltpu.VMEM((nblk, _B), jnp.int32),
            pltpu.VMEM((_K, _B, _DH), jnp.float32),
            pltpu.SemaphoreType.DMA((_K,)),
            pltpu.SemaphoreType.DMA((_K,)),
            pltpu.VMEM_SHARED((_NP, _DH), jnp.float32),
        ],
        compiler_params=pltpu.CompilerParams(use_tc_tiling_on_sc=False),
    )
    def agg_kernel(h_hbm, src_hbm, dst_hbm, out_hbm, srcv, dstv, rows,
                   gsem, ssem, acc):
        c = lax.axis_index("c")
        s = lax.axis_index("s")
        pltpu.sync_copy(src_hbm.at[pl.ds(c * _EBLK + s * nblk, nblk)], srcv)
        pltpu.sync_copy(dst_hbm.at[pl.ds(s * nblk, nblk)], dstv)
        pltpu.sync_copy(h_hbm.at[pl.ds(c * _NP + s * _RPS, _RPS)],
                        acc.at[pl.ds(s * _RPS, _RPS)])
        plsc.subcore_barrier()

        for b in range(_K):
            pltpu.async_copy(h_hbm.at[srcv.at[b]], rows.at[b], gsem.at[b])

        @pl.loop(0, nblk, step=_K)
        def _(j):
            for b in range(_K):
                pltpu.make_async_copy(h_hbm.at[srcv.at[0]], rows.at[b],
                                      gsem.at[b]).wait()
                pltpu.async_copy(rows.at[b], acc.at[dstv.at[j + b]],
                                 ssem.at[b], add=True)
            for b in range(_K):
                pltpu.make_async_copy(rows.at[b], acc.at[dstv.at[0]],
                                      ssem.at[b]).wait()

                @pl.when(j + _K < nblk)
                def _():
                    pltpu.async_copy(h_hbm.at[srcv.at[j + _K + b]],
                                     rows.at[b], gsem.at[b])

        plsc.subcore_barrier()
        pltpu.sync_copy(acc.at[pl.ds(s * _RPS, _RPS)],
                        out_hbm.at[pl.ds(c * _NP + s * _RPS, _RPS)])

    return agg_kernel


_deg = _make_deg_kernel()
_agg = _make_agg_kernel()


def _mm_body(x_ref, w_ref, o_ref):
    o_ref[...] = jnp.dot(x_ref[...], w_ref[...],
                         preferred_element_type=jnp.float32)


def _mm(x, w):
    return pl.pallas_call(
        _mm_body,
        out_shape=jax.ShapeDtypeStruct((x.shape[0], w.shape[1]), jnp.float32),
    )(x, w)


def _scale_body(h_ref, cnt_ref, ha_ref, hb_ref, d_ref):
    c0 = cnt_ref[0:_N, 0:1]
    c1 = cnt_ref[_NP:_NP + _N, 0:1]
    d = lax.rsqrt(1.0 + c0 + c1)
    d_ref[...] = d
    hs = h_ref[...] * d
    ha_ref[0:_N, :] = hs[:, 0:64]
    ha_ref[_NP:_NP + _N, :] = hs[:, 64:128]
    hb_ref[0:_N, :] = hs[:, 128:192]
    hb_ref[_NP:_NP + _N, :] = hs[:, 192:256]


def _scale(h, cnt):
    return pl.pallas_call(
        _scale_body,
        out_shape=(jax.ShapeDtypeStruct((2 * _NP, _DH), jnp.float32),
                   jax.ShapeDtypeStruct((2 * _NP, _DH), jnp.float32),
                   jax.ShapeDtypeStruct((_N, 1), jnp.float32)),
    )(h, cnt)


def _mm2_body(aa_ref, ab_ref, d_ref, b1_ref, w2_ref, o_ref):
    d = d_ref[...]
    h1q0 = jnp.maximum(aa_ref[0:_N, :] * d + b1_ref[0, 0:64], 0.0)
    h1q1 = jnp.maximum(aa_ref[_NP:_NP + _N, :] * d + b1_ref[0, 64:128], 0.0)
    h1q2 = jnp.maximum(ab_ref[0:_N, :] * d + b1_ref[0, 128:192], 0.0)
    h1q3 = jnp.maximum(ab_ref[_NP:_NP + _N, :] * d + b1_ref[0, 192:256], 0.0)
    h2 = (jnp.dot(h1q0, w2_ref[0:64, :], preferred_element_type=jnp.float32)
          + jnp.dot(h1q1, w2_ref[64:128, :], preferred_element_type=jnp.float32)
          + jnp.dot(h1q2, w2_ref[128:192, :], preferred_element_type=jnp.float32)
          + jnp.dot(h1q3, w2_ref[192:256, :], preferred_element_type=jnp.float32))
    h2 = h2 * d
    o_ref[0:_N, :] = h2[:, 0:64]
    o_ref[_NP:_NP + _N, :] = h2[:, 64:]


def _mm2(a1a, a1b, d, b1, w2):
    return pl.pallas_call(
        _mm2_body,
        out_shape=jax.ShapeDtypeStruct((2 * _NP, _DH), jnp.float32),
    )(a1a, a1b, d, b1, w2)


def _fin_body(a_ref, d_ref, b2_ref, o_ref):
    d = d_ref[...]
    lo = a_ref[0:_N, :]
    hi = a_ref[_NP:_NP + _N, :]
    o_ref[...] = jnp.concatenate([lo, hi], axis=1) * d + b2_ref[...]


def _fin(acc2, d, b2):
    return pl.pallas_call(
        _fin_body,
        out_shape=jax.ShapeDtypeStruct((_N, 128), jnp.float32),
    )(acc2, d, b2)


def kernel(x, edge_index, W1, b1, W2, b2):
    src = edge_index[0]
    dst = edge_index[1]
    npad = _EP - _E
    srcp = jnp.concatenate([src, jnp.zeros((npad,), jnp.int32)])
    dstp = jnp.concatenate([dst, jnp.full((npad,), _N, jnp.int32)])
    dst2 = dstp.reshape(_EBLK, _B)
    src2 = jnp.concatenate([srcp, srcp + _NP]).reshape(2 * _EBLK, _B)
    ones16 = jnp.ones((_B, 16), jnp.float32)
    zeros16 = jnp.zeros((_NP, 16), jnp.float32)

    cnt = _deg(dst2, ones16, zeros16)
    h = _mm(x, W1)
    h1a, h1b, d = _scale(h, cnt)
    a1a = _agg(h1a, src2, dst2)
    a1b = _agg(h1b, src2, dst2)
    h2cat = _mm2(a1a, a1b, d, b1.reshape(1, -1), W2)
    acc2 = _agg(h2cat, src2, dst2)
    return _fin(acc2, d, b2.reshape(1, -1))
